# baseline (speedup 1.0000x reference)
import functools

import jax
import jax.numpy as jnp
import numpy as np
from jax import lax
from jax.experimental import pallas as pl
from jax.experimental.pallas import tpu as pltpu
from jax.experimental.pallas import tpu_sc as plsc

N_NODES = 10000
N_PAD = 10240
CAP = 64
E = 320000
D_IN = 128
D_HID = 16
D_OUT = 64

NW = 32
R_TOT = CAP * N_PAD
RW = R_TOT // NW
CHUNK = 80
FIRE = 8
GROUP = CHUNK * FIRE
N_GROUPS = RW // GROUP


def _sc_gather_body(table_hbm, src_hbm, pos_hbm, out_hbm, pos_v, idx_v, buf_v,
                    isem, gsem):
    wid = lax.axis_index("s") * 2 + lax.axis_index("c")
    base = wid * RW
    pltpu.sync_copy(pos_hbm.at[pl.ds(base, RW)], pos_v)

    def group(g, carry):
        row0 = g * GROUP
        ih = []
        for b in range(FIRE):
            ih.append(pltpu.async_copy(
                src_hbm.at[pos_v.at[pl.ds(row0 + b * CHUNK, CHUNK)]],
                idx_v.at[pl.ds(b * CHUNK, CHUNK)],
                isem,
            ))
        for h in ih:
            h.wait()
        gh = []
        for b in range(FIRE):
            gh.append(pltpu.async_copy(
                table_hbm.at[idx_v.at[pl.ds(b * CHUNK, CHUNK)]],
                buf_v.at[pl.ds(b * CHUNK, CHUNK)],
                gsem,
            ))
        for h in gh:
            h.wait()
        pltpu.sync_copy(buf_v, out_hbm.at[pl.ds(base + row0, GROUP)])
        return carry

    lax.fori_loop(0, N_GROUPS, group, 0)


def _sc_gather(table, src_sorted, pos, d):
    fn = pl.kernel(
        _sc_gather_body,
        out_type=jax.ShapeDtypeStruct((R_TOT, d), jnp.float32),
        mesh=plsc.VectorSubcoreMesh(core_axis_name="c", subcore_axis_name="s"),
        scratch_types=[
            pltpu.VMEM((RW,), jnp.int32),
            pltpu.VMEM((GROUP,), jnp.int32),
            pltpu.VMEM((GROUP, d), jnp.float32),
            pltpu.SemaphoreType.DMA,
            pltpu.SemaphoreType.DMA,
        ],
        compiler_params=pltpu.CompilerParams(use_tc_tiling_on_sc=False),
    )
    return fn(table, src_sorted, pos)


def _median_body(g_ref, cnt_ref, k_ref, b_ref, o_ref, *, relu):
    nb = g_ref.shape[1]
    i3 = lax.broadcasted_iota(jnp.int32, (CAP, nb, 128), 0)
    cnt = cnt_ref[...]
    x = jnp.where(i3 < cnt[None, :, :], g_ref[...], jnp.inf)

    size = 2
    while size <= CAP:
        stride = size // 2
        while stride >= 1:
            gsz = CAP // (2 * stride)
            x5 = x.reshape(gsz, 2, stride, nb, 128)
            xp5 = jnp.concatenate([x5[:, 1:2], x5[:, 0:1]], axis=1)
            lo = jnp.minimum(x5, xp5).reshape(CAP, nb, 128)
            hi = jnp.maximum(x5, xp5).reshape(CAP, nb, 128)
            keep_lo = ((i3 & size) == 0) == ((i3 & stride) == 0)
            x = jnp.where(keep_lo, lo, hi)
            stride //= 2
        size *= 2

    kv = k_ref[...]
    acc = jnp.zeros_like(cnt, dtype=jnp.float32)
    for j in range(CAP):
        acc = acc + jnp.where(kv == j, x[j], 0.0)
    med = jnp.where(cnt > 0, acc, 0.0) + b_ref[0:1, :]
    if relu:
        med = jnp.maximum(med, 0.0)
    o_ref[...] = med


def _median_select(g, cnt_pack, k_pack, bias_tile, relu):
    nl = g.shape[1]
    nb = nl // 128
    return pl.pallas_call(
        functools.partial(_median_body, relu=relu),
        grid=(nb,),
        in_specs=[
            pl.BlockSpec((CAP, 128, 128), lambda i: (0, i, 0)),
            pl.BlockSpec((128, 128), lambda i: (i, 0)),
            pl.BlockSpec((128, 128), lambda i: (i, 0)),
            pl.BlockSpec((8, 128), lambda i: (0, 0)),
        ],
        out_specs=pl.BlockSpec((128, 128), lambda i: (i, 0)),
        out_shape=jax.ShapeDtypeStruct((nl, 128), jnp.float32),
    )(g, cnt_pack, k_pack, bias_tile)


def _matmul_body(a_ref, w_ref, o_ref):
    o_ref[...] = jnp.dot(a_ref[...], w_ref[...],
                         preferred_element_type=jnp.float32)


def _matmul(a, w):
    return pl.pallas_call(
        _matmul_body,
        out_shape=jax.ShapeDtypeStruct((a.shape[0], w.shape[1]), jnp.float32),
    )(a, w)


def _segment_median_sorted(vals_sorted_by_dst, dst_sorted, start, counts):
    k = jnp.where(counts > 0, (counts - 1) // 2, 0)
    pos = jnp.clip(start + k, 0, E - 1)

    def col(colv):
        order = jnp.argsort(colv, stable=True)
        key2 = dst_sorted[order]
        order2 = jnp.argsort(key2, stable=True)
        return colv[order][order2][pos]

    med = jax.vmap(col, in_axes=1, out_axes=1)(vals_sorted_by_dst)
    return jnp.where((counts > 0)[:, None], med, 0.0)


def _slow_path(x, src_sorted, dst_sorted, start, counts, W1, b1, W2, b2):
    h = x @ W1
    m1 = _segment_median_sorted(h[src_sorted], dst_sorted, start, counts) + b1
    h2 = jnp.maximum(m1, 0.0) @ W2
    m2 = _segment_median_sorted(h2[src_sorted], dst_sorted, start, counts) + b2
    return m2


def _pack(v, reps):
    return jnp.repeat(v, reps).reshape(-1, 128)


def _fast_path(x, src_sorted, pos, cnt1, k1p, cnt2, k2p, b1t, b2t, W1, W2):
    h1 = _matmul(x, W1)
    g1 = _sc_gather(h1, src_sorted, pos, D_HID)
    g1 = g1.reshape(CAP, N_PAD * D_HID // 128, 128)
    m1 = _median_select(g1, cnt1, k1p, b1t, relu=True)
    h2 = _matmul(m1.reshape(N_PAD, D_HID), W2)
    g2 = _sc_gather(h2, src_sorted, pos, D_OUT)
    g2 = g2.reshape(CAP, N_PAD * D_OUT // 128, 128)
    m2 = _median_select(g2, cnt2, k2p, b2t, relu=False)
    return m2.reshape(N_PAD, D_OUT)[:N_NODES]


def kernel(x, edge_index, W1, b1, W2, b2):
    src = edge_index[0]
    dst = edge_index[1]
    packed = jnp.sort((dst << 14) | src)
    src_sorted = packed & 0x3FFF
    counts = jnp.bincount(dst, length=N_NODES)
    start = jnp.concatenate(
        [jnp.zeros((1,), counts.dtype), jnp.cumsum(counts)[:-1]])

    counts_p = jnp.pad(counts, (0, N_PAD - N_NODES))
    start_p = jnp.pad(start, (0, N_PAD - N_NODES), constant_values=E - 1)
    k_p = jnp.where(counts_p > 0, (counts_p - 1) // 2, 0)

    pos = jnp.clip(start_p[None, :] + jnp.arange(CAP, dtype=jnp.int32)[:, None],
                   0, E - 1).reshape(-1).astype(jnp.int32)

    cnt1 = _pack(counts_p, D_HID)
    k1p = _pack(k_p, D_HID)
    cnt2 = _pack(counts_p, D_OUT)
    k2p = _pack(k_p, D_OUT)
    b1t = jnp.tile(b1, (8, 128 // D_HID))
    b2t = jnp.tile(b2, (8, 128 // D_OUT))

    fits = jnp.max(counts) <= CAP
    return lax.cond(
        fits,
        lambda: _fast_path(x, src_sorted, pos, cnt1, k1p, cnt2, k2p, b1t, b2t,
                           W1, W2) + 0.0,
        lambda: _slow_path(x, src_sorted, packed >> 14, start, counts, W1, b1,
                           W2, b2),
    )


# speedup vs baseline: 1.2470x; 1.2470x over previous
import functools

import jax
import jax.numpy as jnp
import numpy as np
from jax import lax
from jax.experimental import pallas as pl
from jax.experimental.pallas import tpu as pltpu
from jax.experimental.pallas import tpu_sc as plsc

N_NODES = 10000
N_PAD = 10240
CAP = 64
E = 320000
D_IN = 128
D_HID = 16
D_OUT = 64

NW = 32
R_TOT = CAP * N_PAD
EW = E // NW
CHUNK = 80
FIRE = 5
GROUP = CHUNK * FIRE
N_GROUPS = EW // GROUP


def _sc_gather_body(table_hbm, src_hbm, spos_hbm, out_hbm, *refs):
    ids_v = refs[0:FIRE]
    pos_v = refs[FIRE:2 * FIRE]
    buf_v = refs[2 * FIRE]
    lsem, gsem, ssem = refs[2 * FIRE + 1:2 * FIRE + 4]
    wid = lax.axis_index("s") * 2 + lax.axis_index("c")
    base = wid * EW

    def group(g, carry):
        e0 = base + g * GROUP
        lh = []
        for b in range(FIRE):
            lh.append(pltpu.async_copy(
                src_hbm.at[pl.ds(e0 + b * CHUNK, CHUNK)], ids_v[b], lsem))
            lh.append(pltpu.async_copy(
                spos_hbm.at[pl.ds(e0 + b * CHUNK, CHUNK)], pos_v[b], lsem))
        for h in lh:
            h.wait()
        gh = []
        for b in range(FIRE):
            gh.append(pltpu.async_copy(
                table_hbm.at[ids_v[b]],
                buf_v.at[pl.ds(b * CHUNK, CHUNK)], gsem))
        for h in gh:
            h.wait()
        sh = []
        for b in range(FIRE):
            sh.append(pltpu.async_copy(
                buf_v.at[pl.ds(b * CHUNK, CHUNK)],
                out_hbm.at[pos_v[b]], ssem))
        for h in sh:
            h.wait()
        return carry

    lax.fori_loop(0, N_GROUPS, group, 0)


def _sc_gather(table, src_sorted, spos, d):
    fn = pl.kernel(
        _sc_gather_body,
        out_type=jax.ShapeDtypeStruct((R_TOT, d), jnp.float32),
        mesh=plsc.VectorSubcoreMesh(core_axis_name="c", subcore_axis_name="s"),
        scratch_types=(
            [pltpu.VMEM((CHUNK,), jnp.int32) for _ in range(2 * FIRE)]
            + [pltpu.VMEM((GROUP, d), jnp.float32),
               pltpu.SemaphoreType.DMA,
               pltpu.SemaphoreType.DMA,
               pltpu.SemaphoreType.DMA]
        ),
        compiler_params=pltpu.CompilerParams(use_tc_tiling_on_sc=False),
    )
    return fn(table, src_sorted, spos)


def _median_body(g_ref, cnt_ref, k_ref, b_ref, o_ref, *, relu):
    nb = g_ref.shape[1]
    i3 = lax.broadcasted_iota(jnp.int32, (CAP, nb, 128), 0)
    cnt = cnt_ref[...]
    x = jnp.where(i3 < cnt[None, :, :], g_ref[...], jnp.inf)

    size = 2
    while size <= CAP:
        stride = size // 2
        while stride >= 1:
            gsz = CAP // (2 * stride)
            x5 = x.reshape(gsz, 2, stride, nb, 128)
            xp5 = jnp.concatenate([x5[:, 1:2], x5[:, 0:1]], axis=1)
            lo = jnp.minimum(x5, xp5).reshape(CAP, nb, 128)
            hi = jnp.maximum(x5, xp5).reshape(CAP, nb, 128)
            keep_lo = ((i3 & size) == 0) == ((i3 & stride) == 0)
            x = jnp.where(keep_lo, lo, hi)
            stride //= 2
        size *= 2

    kv = k_ref[...]
    acc = jnp.zeros_like(cnt, dtype=jnp.float32)
    for j in range(CAP):
        acc = acc + jnp.where(kv == j, x[j], 0.0)
    med = jnp.where(cnt > 0, acc, 0.0) + b_ref[0:1, :]
    if relu:
        med = jnp.maximum(med, 0.0)
    o_ref[...] = med


def _median_select(g, cnt_pack, k_pack, bias_tile, relu):
    nl = g.shape[1]
    nb = nl // 128
    return pl.pallas_call(
        functools.partial(_median_body, relu=relu),
        grid=(nb,),
        in_specs=[
            pl.BlockSpec((CAP, 128, 128), lambda i: (0, i, 0)),
            pl.BlockSpec((128, 128), lambda i: (i, 0)),
            pl.BlockSpec((128, 128), lambda i: (i, 0)),
            pl.BlockSpec((8, 128), lambda i: (0, 0)),
        ],
        out_specs=pl.BlockSpec((128, 128), lambda i: (i, 0)),
        out_shape=jax.ShapeDtypeStruct((nl, 128), jnp.float32),
    )(g, cnt_pack, k_pack, bias_tile)


def _matmul_body(a_ref, w_ref, o_ref):
    o_ref[...] = jnp.dot(a_ref[...], w_ref[...],
                         preferred_element_type=jnp.float32)


def _matmul(a, w):
    return pl.pallas_call(
        _matmul_body,
        out_shape=jax.ShapeDtypeStruct((a.shape[0], w.shape[1]), jnp.float32),
    )(a, w)


def _segment_median_sorted(vals_sorted_by_dst, dst_sorted, start, counts):
    k = jnp.where(counts > 0, (counts - 1) // 2, 0)
    pos = jnp.clip(start + k, 0, E - 1)

    def col(colv):
        order = jnp.argsort(colv, stable=True)
        key2 = dst_sorted[order]
        order2 = jnp.argsort(key2, stable=True)
        return colv[order][order2][pos]

    med = jax.vmap(col, in_axes=1, out_axes=1)(vals_sorted_by_dst)
    return jnp.where((counts > 0)[:, None], med, 0.0)


def _slow_path(x, src_sorted, dst_sorted, start, counts, W1, b1, W2, b2):
    h = x @ W1
    m1 = _segment_median_sorted(h[src_sorted], dst_sorted, start, counts) + b1
    h2 = jnp.maximum(m1, 0.0) @ W2
    m2 = _segment_median_sorted(h2[src_sorted], dst_sorted, start, counts) + b2
    return m2


def _pack(v, reps):
    return jnp.repeat(v, reps).reshape(-1, 128)


def _fast_path(x, src_sorted, spos, cnt1, k1p, cnt2, k2p, b1t, b2t, W1, W2):
    h1 = _matmul(x, W1)
    g1 = _sc_gather(h1, src_sorted, spos, D_HID)
    g1 = g1.reshape(CAP, N_PAD * D_HID // 128, 128)
    m1 = _median_select(g1, cnt1, k1p, b1t, relu=True)
    h2 = _matmul(m1.reshape(N_PAD, D_HID), W2)
    g2 = _sc_gather(h2, src_sorted, spos, D_OUT)
    g2 = g2.reshape(CAP, N_PAD * D_OUT // 128, 128)
    m2 = _median_select(g2, cnt2, k2p, b2t, relu=False)
    return m2.reshape(N_PAD, D_OUT)[:N_NODES]


def kernel(x, edge_index, W1, b1, W2, b2):
    src = edge_index[0]
    dst = edge_index[1]
    packed = jnp.sort((dst << 14) | src)
    src_sorted = packed & 0x3FFF
    counts = jnp.bincount(dst, length=N_NODES)
    start = jnp.concatenate(
        [jnp.zeros((1,), counts.dtype), jnp.cumsum(counts)[:-1]])

    counts_p = jnp.pad(counts, (0, N_PAD - N_NODES))
    k_p = jnp.where(counts_p > 0, (counts_p - 1) // 2, 0)

    seg = packed >> 14
    eidx = jnp.arange(E, dtype=jnp.int32)
    newrun = jnp.concatenate(
        [jnp.ones((1,), jnp.bool_), seg[1:] != seg[:-1]])
    runstart = lax.associative_scan(jnp.maximum, jnp.where(newrun, eidx, 0))
    spos = jnp.clip((eidx - runstart) * N_PAD + seg, 0, R_TOT - 1)

    cnt1 = _pack(counts_p, D_HID)
    k1p = _pack(k_p, D_HID)
    cnt2 = _pack(counts_p, D_OUT)
    k2p = _pack(k_p, D_OUT)
    b1t = jnp.tile(b1, (8, 128 // D_HID))
    b2t = jnp.tile(b2, (8, 128 // D_OUT))

    fits = jnp.max(counts) <= CAP
    return lax.cond(
        fits,
        lambda: _fast_path(x, src_sorted, spos, cnt1, k1p, cnt2, k2p, b1t, b2t,
                           W1, W2) + 0.0,
        lambda: _slow_path(x, src_sorted, seg, start, counts, W1, b1, W2, b2),
    )
